# baseline (speedup 1.0000x reference)
import functools

import jax
import jax.numpy as jnp
from jax import lax
from jax.experimental import pallas as pl
from jax.experimental.pallas import tpu as pltpu
from jax.experimental.pallas import tpu_sc as plsc

VOCAB = 100000
Q_HEADS = 16
LATENT_DIM = 64
BATCH = 64
CONTEXT = 2048

NC = 2
NS = 16
NW = NC * NS
POS_PER_W = CONTEXT // NW
NB = 16
N_BLK = BATCH // NB
QROWS_PER_W = 8


def _lane_swap(x):
    perm = jax.lax.iota(jnp.int32, 16) ^ 1
    dnums = lax.GatherDimensionNumbers(
        offset_dims=(), collapsed_slice_dims=(0,), start_index_map=(0,))
    return lax.gather(x, perm[:, None], dnums, (1,),
                      mode=lax.GatherScatterMode.PROMISE_IN_BOUNDS)


def _rope_cache():
    pos = jnp.arange(CONTEXT + 1, dtype=jnp.float32)
    inv_freq = 1.0 / (10000.0 ** (
        jnp.arange(0, LATENT_DIM, 2, dtype=jnp.float32) / LATENT_DIM))
    freqs = pos[:, None] * inv_freq[None, :]
    emb = jnp.repeat(freqs, 2, axis=-1)
    alt = jnp.where(jnp.arange(LATENT_DIM) % 2 == 0, -1.0, 1.0).astype(jnp.float32)
    return jnp.cos(emb), jnp.sin(emb) * alt


def _body(ctx_hbm, nt_hbm, qtab_hbm, ltab_hbm, cos_hbm, ss_hbm,
          cosq_hbm, ssq_hbm, out_lat, out_q,
          cos_v, ss_v, tok_v, rows_v, nt_v, qrows_v, cq_v, sq_v, sem):
    wid = lax.axis_index("s") * NC + lax.axis_index("c")
    pos0 = wid * POS_PER_W

    pltpu.sync_copy(cos_hbm.at[pl.ds(pos0, POS_PER_W)], cos_v)
    pltpu.sync_copy(ss_hbm.at[pl.ds(pos0, POS_PER_W)], ss_v)
    tok_copies = [
        pltpu.async_copy(ctx_hbm.at[pl.ds(b * CONTEXT + pos0, POS_PER_W)],
                         tok_v.at[b], sem)
        for b in range(BATCH)
    ]
    for c in tok_copies:
        c.wait()

    for blk in range(N_BLK):
        b0 = blk * NB
        copies = [
            pltpu.async_copy(ltab_hbm.at[tok_v.at[b0 + bl]], rows_v.at[bl], sem)
            for bl in range(NB)
        ]
        for c in copies:
            c.wait()

        def r_body(r, carry):
            cs = [cos_v[r, pl.ds(16 * j, 16)] for j in range(4)]
            sg = [ss_v[r, pl.ds(16 * j, 16)] for j in range(4)]

            def b_body(bl, inner):
                for j in range(4):
                    x = rows_v[bl, r, pl.ds(16 * j, 16)]
                    rows_v[bl, r, pl.ds(16 * j, 16)] = (
                        x * cs[j] + _lane_swap(x) * sg[j])
                return inner

            lax.fori_loop(0, NB, b_body, carry, unroll=2)
            return carry

        lax.fori_loop(0, POS_PER_W, r_body, 0)

        copies = [
            pltpu.async_copy(
                rows_v.at[bl],
                out_lat.at[pl.ds((b0 + bl) * CONTEXT + pos0, POS_PER_W)],
                sem)
            for bl in range(NB)
        ]
        for c in copies:
            c.wait()

    @pl.when(wid < BATCH // QROWS_PER_W)
    def _():
        pltpu.sync_copy(nt_hbm.at[pl.ds(wid * QROWS_PER_W, QROWS_PER_W)], nt_v)
        pltpu.async_copy(qtab_hbm.at[nt_v], qrows_v, sem).wait()
        pltpu.sync_copy(cosq_hbm, cq_v)
        pltpu.sync_copy(ssq_hbm, sq_v)
        cqs = [cq_v[pl.ds(16 * j, 16)] for j in range(4)]
        sqs = [sq_v[pl.ds(16 * j, 16)] for j in range(4)]

        def qr_body(r, carry):
            def qh_body(h, inner):
                base = h * LATENT_DIM
                for j in range(4):
                    x = qrows_v[r, pl.ds(base + 16 * j, 16)]
                    qrows_v[r, pl.ds(base + 16 * j, 16)] = (
                        x * cqs[j] + _lane_swap(x) * sqs[j])
                return inner

            lax.fori_loop(0, Q_HEADS, qh_body, carry)
            return carry

        lax.fori_loop(0, QROWS_PER_W, qr_body, 0)
        pltpu.sync_copy(qrows_v,
                        out_q.at[pl.ds(wid * QROWS_PER_W, QROWS_PER_W)])


@jax.jit
def kernel(context_tokens, next_tokens, q_table, latent_table):
    cos_t, ss_t = _rope_cache()
    cos_ctx = cos_t[:CONTEXT]
    ss_ctx = ss_t[:CONTEXT]
    cos_q = cos_t[CONTEXT]
    ss_q = ss_t[CONTEXT]

    mesh = plsc.VectorSubcoreMesh(core_axis_name="c", subcore_axis_name="s")
    run = functools.partial(
        pl.kernel,
        mesh=mesh,
        compiler_params=pltpu.CompilerParams(use_tc_tiling_on_sc=False),
        out_type=[
            jax.ShapeDtypeStruct((BATCH * CONTEXT, LATENT_DIM), jnp.float32),
            jax.ShapeDtypeStruct((BATCH, Q_HEADS * LATENT_DIM), jnp.float32),
        ],
        scratch_types=[
            pltpu.VMEM((POS_PER_W, LATENT_DIM), jnp.float32),
            pltpu.VMEM((POS_PER_W, LATENT_DIM), jnp.float32),
            pltpu.VMEM((BATCH, POS_PER_W), jnp.int32),
            pltpu.VMEM((NB, POS_PER_W, LATENT_DIM), jnp.float32),
            pltpu.VMEM((QROWS_PER_W,), jnp.int32),
            pltpu.VMEM((QROWS_PER_W, Q_HEADS * LATENT_DIM), jnp.float32),
            pltpu.VMEM((LATENT_DIM,), jnp.float32),
            pltpu.VMEM((LATENT_DIM,), jnp.float32),
            pltpu.SemaphoreType.DMA,
        ],
    )(_body)
    out_lat, out_q = run(
        context_tokens.astype(jnp.int32).reshape(-1),
        next_tokens.astype(jnp.int32),
        q_table, latent_table, cos_ctx, ss_ctx, cos_q, ss_q)
    q = out_q.reshape(BATCH, Q_HEADS, 1, LATENT_DIM)
    latent = out_lat.reshape(BATCH, CONTEXT, LATENT_DIM)
    return (q, latent)
